# baseline (speedup 1.0000x reference)
import functools

import jax
import jax.numpy as jnp
from jax import lax
from jax.experimental import pallas as pl
from jax.experimental.pallas import tpu as pltpu
from jax.experimental.pallas import tpu_sc as plsc

N = 10000
E = 160000
D = 256
H = 128
NC = 2
NS = 16
EPS = E // NS
BLK = 80
NBLK = EPS // BLK
NPAD = 10240
NPS = NPAD // NS
NDEG = 10240
F32 = jnp.float32

@functools.lru_cache(None)
def _get_mesh():
    return plsc.VectorSubcoreMesh(core_axis_name="c", subcore_axis_name="s",
                                  num_cores=NC, num_subcores=NS)


def _ln_rows(z, g, b, eps=1e-5):
    mu = jnp.mean(z, axis=-1, keepdims=True)
    var = jnp.mean((z - mu) ** 2, axis=-1, keepdims=True)
    return (z - mu) * jax.lax.rsqrt(var + eps) * g + b


@functools.lru_cache(None)
def _make_sc_agg(with_deg):
    out_type = [jax.ShapeDtypeStruct((NPAD, H), F32),
                jax.ShapeDtypeStruct((NPAD, H), F32)]
    if with_deg:
        out_type.append(jax.ShapeDtypeStruct((NDEG,), F32))

    scratch = dict(
        idx_s0=pltpu.VMEM((BLK,), jnp.int32),
        idx_s1=pltpu.VMEM((BLK,), jnp.int32),
        idx_d0=pltpu.VMEM((BLK,), jnp.int32),
        idx_d1=pltpu.VMEM((BLK,), jnp.int32),
        w_v0=pltpu.VMEM((BLK,), F32),
        w_v1=pltpu.VMEM((BLK,), F32),
        rows0=pltpu.VMEM((BLK, H), F32),
        rows1=pltpu.VMEM((BLK, H), F32),
        ones_v=pltpu.VMEM((BLK,), F32),
        agg_sp=pltpu.VMEM_SHARED((NPAD, H), F32),
        deg_sp=pltpu.VMEM_SHARED((NDEG,), F32),
        isem0=pltpu.SemaphoreType.DMA,
        isem1=pltpu.SemaphoreType.DMA,
        jsem0=pltpu.SemaphoreType.DMA,
        jsem1=pltpu.SemaphoreType.DMA,
        gsem0=pltpu.SemaphoreType.DMA,
        gsem1=pltpu.SemaphoreType.DMA,
        ssem0=pltpu.SemaphoreType.DMA,
        ssem1=pltpu.SemaphoreType.DMA,
        dsem0=pltpu.SemaphoreType.DMA,
        dsem1=pltpu.SemaphoreType.DMA,
    )

    @functools.partial(pl.kernel, mesh=_get_mesh(), out_type=out_type,
                       scratch_types=scratch)
    def sc_agg(hh0, hh1, src, dst, w, zrows, zdeg, ones, *refs,
               idx_s0, idx_s1, idx_d0, idx_d1, w_v0, w_v1, rows0, rows1,
               ones_v, agg_sp, deg_sp, isem0, isem1, jsem0, jsem1,
               gsem0, gsem1, ssem0, ssem1, dsem0, dsem1):
        if with_deg:
            agg0_o, agg1_o, deg_o = refs[0], refs[1], refs[2]
        else:
            agg0_o, agg1_o = refs[0], refs[1]
            deg_o = None

        c = lax.axis_index("c")
        s = lax.axis_index("s")
        idx_s = (idx_s0, idx_s1)
        idx_d = (idx_d0, idx_d1)
        w_v = (w_v0, w_v1)
        rows = (rows0, rows1)
        isem = (isem0, isem1)
        jsem = (jsem0, jsem1)
        gsem = (gsem0, gsem1)
        ssem = (ssem0, ssem1)
        dsem = (dsem0, dsem1)

        def run(tbl, agg_out, do_deg):
            pltpu.sync_copy(zrows, agg_sp.at[pl.ds(s * NPS, NPS)])
            if do_deg:
                pltpu.sync_copy(zdeg.at[pl.ds(s * 640, 640)],
                                deg_sp.at[pl.ds(s * 640, 640)])
                pltpu.sync_copy(ones, ones_v)
            plsc.subcore_barrier()

            def start_idx_sw(b, m):
                base = s * EPS + b * BLK
                pltpu.async_copy(src.at[pl.ds(base, BLK)], idx_s[m], isem[m])
                pltpu.async_copy(w.at[pl.ds(base, BLK)], w_v[m],
                                 isem[m])

            def wait_idx_sw(m):
                pltpu.make_async_copy(src.at[pl.ds(0, BLK)], idx_s[m],
                                      isem[m]).wait()
                pltpu.make_async_copy(w.at[pl.ds(0, BLK)], w_v[m],
                                      isem[m]).wait()

            def start_idx_d(b, m):
                base = s * EPS + b * BLK
                pltpu.async_copy(dst.at[pl.ds(base, BLK)], idx_d[m], jsem[m])

            def wait_idx_d(m):
                pltpu.make_async_copy(dst.at[pl.ds(0, BLK)], idx_d[m],
                                      jsem[m]).wait()

            def start_gather(m):
                pltpu.async_copy(tbl.at[idx_s[m]], rows[m], gsem[m])

            def wait_gather(m):
                pltpu.make_async_copy(tbl.at[idx_s[m]], rows[m],
                                      gsem[m]).wait()

            def start_scatter(m):
                pltpu.async_copy(rows[m], agg_sp.at[idx_d[m]], ssem[m],
                                 add=True)
                if do_deg:
                    pltpu.async_copy(ones_v, deg_sp.at[idx_d[m]], dsem[m],
                                     add=True)

            def wait_scatter(m):
                pltpu.make_async_copy(rows[m], agg_sp.at[idx_d[m]],
                                      ssem[m]).wait()
                if do_deg:
                    pltpu.make_async_copy(ones_v, deg_sp.at[idx_d[m]],
                                          dsem[m]).wait()

            def step(b, m):
                mo = 1 - m
                wait_gather(m)

                @pl.when(b + 1 < NBLK)
                def _():
                    @pl.when(b >= 1)
                    def _():
                        wait_scatter(mo)
                        start_idx_d(b + 1, mo)
                    wait_idx_sw(mo)
                    start_gather(mo)

                @plsc.parallel_loop(0, BLK, step=1, unroll=4)
                def _(i):
                    g = (i // 16) * 16
                    wchunk = w_v[m][pl.ds(g, 16)]
                    w16 = wchunk.at[jnp.zeros((16,), jnp.int32)
                                    + (i - g)].get(mode="promise_in_bounds")
                    for j in range(H // 16):
                        sl = pl.ds(j * 16, 16)
                        rows[m][i, sl] = rows[m][i, sl] * w16

                @pl.when(b + 2 < NBLK)
                def _():
                    start_idx_sw(b + 2, m)

                wait_idx_d(m)
                start_scatter(m)

            start_idx_sw(0, 0)
            start_idx_sw(1, 1)
            start_idx_d(0, 0)
            start_idx_d(1, 1)
            wait_idx_sw(0)
            start_gather(0)

            def pair_body(k, carry):
                step(2 * k, 0)
                step(2 * k + 1, 1)
                return carry
            lax.fori_loop(0, NBLK // 2, pair_body, 0)
            if NBLK % 2:
                step(NBLK - 1, 0)
            wait_scatter(1)
            wait_scatter(0)

            plsc.subcore_barrier()
            pltpu.sync_copy(agg_sp.at[pl.ds(s * NPS, NPS)],
                            agg_out.at[pl.ds(s * NPS, NPS)])
            if do_deg:
                pltpu.sync_copy(deg_sp.at[pl.ds(s * 640, 640)],
                                deg_o.at[pl.ds(s * 640, 640)])

        @pl.when(c == 0)
        def _():
            run(hh0, agg0_o, with_deg)

        @pl.when(c == 1)
        def _():
            run(hh1, agg1_o, False)

    return sc_agg


def _sc_agg_deg(*args):
    return _make_sc_agg(True)(*args)


def _sc_agg(*args):
    return _make_sc_agg(False)(*args)


@functools.lru_cache(None)
def _make_sc_pred():
    BLKP = 40
    EPW = E // (NC * NS)
    NBLKP = EPW // BLKP
    HP = H

    @functools.partial(
        pl.kernel, mesh=_get_mesh(),
        out_type=[jax.ShapeDtypeStruct((E, HP), jnp.int32),
                  jax.ShapeDtypeStruct((E, HP), jnp.int32)],
        scratch_types=dict(
            idx_s0=pltpu.VMEM((BLKP,), jnp.int32),
            idx_s1=pltpu.VMEM((BLKP,), jnp.int32),
            idx_d0=pltpu.VMEM((BLKP,), jnp.int32),
            idx_d1=pltpu.VMEM((BLKP,), jnp.int32),
            bufa0=pltpu.VMEM((BLKP, HP), jnp.int32),
            bufa1=pltpu.VMEM((BLKP, HP), jnp.int32),
            bufb0=pltpu.VMEM((BLKP, HP), jnp.int32),
            bufb1=pltpu.VMEM((BLKP, HP), jnp.int32),
            isem0=pltpu.SemaphoreType.DMA,
            isem1=pltpu.SemaphoreType.DMA,
            gsem0=pltpu.SemaphoreType.DMA,
            gsem1=pltpu.SemaphoreType.DMA,
            osem0=pltpu.SemaphoreType.DMA,
            osem1=pltpu.SemaphoreType.DMA,
        ),
    )
    def sc_pred(ta, tb, src, dst, xa_o, xb_o, *,
                idx_s0, idx_s1, idx_d0, idx_d1, bufa0, bufa1, bufb0, bufb1,
                isem0, isem1, gsem0, gsem1, osem0, osem1):
        c = lax.axis_index("c")
        s = lax.axis_index("s")
        wid = s * NC + c
        idx_s = (idx_s0, idx_s1)
        idx_d = (idx_d0, idx_d1)
        bufa = (bufa0, bufa1)
        bufb = (bufb0, bufb1)
        isem = (isem0, isem1)
        gsem = (gsem0, gsem1)
        osem = (osem0, osem1)

        def start_idx(b, m):
            base = wid * EPW + b * BLKP
            pltpu.async_copy(src.at[pl.ds(base, BLKP)], idx_s[m], isem[m])
            pltpu.async_copy(dst.at[pl.ds(base, BLKP)], idx_d[m], isem[m])

        def wait_idx(m):
            pltpu.make_async_copy(src.at[pl.ds(0, BLKP)], idx_s[m],
                                  isem[m]).wait()
            pltpu.make_async_copy(dst.at[pl.ds(0, BLKP)], idx_d[m],
                                  isem[m]).wait()

        def start_gather(m):
            pltpu.async_copy(ta.at[idx_s[m]], bufa[m], gsem[m])
            pltpu.async_copy(tb.at[idx_d[m]], bufb[m], gsem[m])

        def wait_gather(m):
            pltpu.make_async_copy(ta.at[idx_s[m]], bufa[m], gsem[m]).wait()
            pltpu.make_async_copy(tb.at[idx_d[m]], bufb[m], gsem[m]).wait()

        def start_out(b, m):
            base = wid * EPW + b * BLKP
            pltpu.async_copy(bufa[m], xa_o.at[pl.ds(base, BLKP)], osem[m])
            pltpu.async_copy(bufb[m], xb_o.at[pl.ds(base, BLKP)], osem[m])

        def wait_out(b, m):
            base = wid * EPW + b * BLKP
            pltpu.make_async_copy(bufa[m], xa_o.at[pl.ds(base, BLKP)],
                                  osem[m]).wait()
            pltpu.make_async_copy(bufb[m], xb_o.at[pl.ds(base, BLKP)],
                                  osem[m]).wait()

        def step(b, m):
            mo = 1 - m
            wait_gather(m)
            start_out(b, m)

            @pl.when(b + 1 < NBLKP)
            def _():
                wait_idx(mo)

                @pl.when(b >= 1)
                def _():
                    wait_out(b - 1, mo)
                start_gather(mo)

            @pl.when(b + 2 < NBLKP)
            def _():
                start_idx(b + 2, m)

        start_idx(0, 0)
        start_idx(1, 1)
        wait_idx(0)
        start_gather(0)

        def pair_body(k, carry):
            step(2 * k, 0)
            step(2 * k + 1, 1)
            return carry
        lax.fori_loop(0, NBLKP // 2, pair_body, 0)
        if NBLKP % 2:
            step(NBLKP - 1, 0)
        wait_out(NBLKP - 2, 1)
        wait_out(NBLKP - 1, 0)

    return sc_pred


def _sc_pred(*args):
    return _make_sc_pred()(*args)


BT = 1000
BE = 2000


def _full2(shape):
    return pl.BlockSpec(shape, lambda i: (0, 0))


def _tc_proj_body(h_ref, w0t, w1t, c0, c1, g0, g1, be0, be1, o0, o1):
    x = h_ref[...]
    for (lo, wt, cc, gg, bb, oo) in ((0, w0t, c0, g0, be0, o0),
                                     (H, w1t, c1, g1, be1, o1)):
        z = jnp.dot(x[:, lo:lo + H], wt[...],
                    preferred_element_type=F32) + cc[...]
        z = _ln_rows(z, gg[...], bb[...])
        oo[...] = jnp.maximum(z, 0.0)


def _tc_proj(h, w0t, w1t, c0, c1, g0, g1, be0, be1):
    grid = (N // BT,)
    return pl.pallas_call(
        _tc_proj_body,
        grid=grid,
        in_specs=[pl.BlockSpec((BT, D), lambda i: (i, 0)),
                  _full2((H, H)), _full2((H, H)),
                  _full2((1, H)), _full2((1, H)),
                  _full2((1, H)), _full2((1, H)),
                  _full2((1, H)), _full2((1, H))],
        out_specs=[pl.BlockSpec((BT, H), lambda i: (i, 0)),
                   pl.BlockSpec((BT, H), lambda i: (i, 0))],
        out_shape=[jax.ShapeDtypeStruct((N, H), F32),
                   jax.ShapeDtypeStruct((N, H), F32)],
    )(h, w0t, w1t, c0, c1, g0, g1, be0, be1)


def _tc_layer_body(h0, h1, a0, a1, deg, wst, wnt, bs, g, be, o0, o1):
    hcat = jnp.concatenate([h0[...], h1[...]], axis=1)
    dd = jnp.maximum(deg[...], 1.0)
    mean = jnp.concatenate([a0[...], a1[...]], axis=1) / dd
    rst = (jnp.dot(hcat, wst[...], preferred_element_type=F32) + bs[...]
           + jnp.dot(mean, wnt[...], preferred_element_type=F32))
    rst = jnp.maximum(rst, 0.0)
    z = _ln_rows(rst, g[...], be[...])
    o0[...] = z[:, :H]
    o1[...] = z[:, H:]


def _tc_layer(h0, h1, a0, a1, deg, wst, wnt, bs, g, be):
    grid = (N // BT,)
    bspec = pl.BlockSpec((BT, H), lambda i: (i, 0))
    return pl.pallas_call(
        _tc_layer_body,
        grid=grid,
        in_specs=[bspec, bspec, bspec, bspec,
                  pl.BlockSpec((BT, 1), lambda i: (i, 0)),
                  _full2((D, D)), _full2((D, D)),
                  _full2((1, D)), _full2((1, D)), _full2((1, D))],
        out_specs=[bspec, bspec],
        out_shape=[jax.ShapeDtypeStruct((N, H), F32),
                   jax.ShapeDtypeStruct((N, H), F32)],
    )(h0, h1, a0, a1, deg, wst, wnt, bs, g, be)


def _tc_layerp_body(h0, h1, a0, a1, deg, wst, wnt, bs, g, be,
                    w1at, w1bt, b1, o0, o1, ao, bo):
    hcat = jnp.concatenate([h0[...], h1[...]], axis=1)
    dd = jnp.maximum(deg[...], 1.0)
    mean = jnp.concatenate([a0[...], a1[...]], axis=1) / dd
    rst = (jnp.dot(hcat, wst[...], preferred_element_type=F32) + bs[...]
           + jnp.dot(mean, wnt[...], preferred_element_type=F32))
    rst = jnp.maximum(rst, 0.0)
    z = _ln_rows(rst, g[...], be[...])
    o0[...] = z[:, :H]
    o1[...] = z[:, H:]
    aa = jnp.dot(z, w1at[...], preferred_element_type=F32) + b1[...]
    bb = jnp.dot(z, w1bt[...], preferred_element_type=F32)
    ao[...] = aa.astype(jnp.bfloat16)
    bo[...] = bb.astype(jnp.bfloat16)


def _tc_layerp(h0, h1, a0, a1, deg, wst, wnt, bs, g, be, w1at, w1bt, b1):
    btp = 2000
    grid = (N // btp,)
    bspec = pl.BlockSpec((btp, H), lambda i: (i, 0))
    return pl.pallas_call(
        _tc_layerp_body,
        grid=grid,
        in_specs=[bspec, bspec, bspec, bspec,
                  pl.BlockSpec((btp, 1), lambda i: (i, 0)),
                  _full2((D, D)), _full2((D, D)),
                  _full2((1, D)), _full2((1, D)), _full2((1, D)),
                  _full2((D, D)), _full2((D, D)), _full2((1, D))],
        out_specs=[bspec, bspec,
                   pl.BlockSpec((btp, D), lambda i: (i, 0)),
                   pl.BlockSpec((btp, D), lambda i: (i, 0))],
        out_shape=([jax.ShapeDtypeStruct((N, H), F32)] * 2
                   + [jax.ShapeDtypeStruct((N, D), jnp.bfloat16)] * 2),
    )(h0, h1, a0, a1, deg, wst, wnt, bs, g, be, w1at, w1bt, b1)


def _tc_nodemm_body(h0, h1, w1at, w1bt, b1, a0, a1, b0o, b1o):
    hcat = jnp.concatenate([h0[...], h1[...]], axis=1)
    a = jnp.dot(hcat, w1at[...], preferred_element_type=F32) + b1[...]
    b = jnp.dot(hcat, w1bt[...], preferred_element_type=F32)
    a0[...] = a[:, :H]
    a1[...] = a[:, H:]
    b0o[...] = b[:, :H]
    b1o[...] = b[:, H:]


def _tc_nodemm(h0, h1, w1at, w1bt, b1):
    grid = (N // BT,)
    bspec = pl.BlockSpec((BT, H), lambda i: (i, 0))
    return pl.pallas_call(
        _tc_nodemm_body,
        grid=grid,
        in_specs=[bspec, bspec, _full2((D, D)), _full2((D, D)),
                  _full2((1, D))],
        out_specs=[bspec, bspec, bspec, bspec],
        out_shape=[jax.ShapeDtypeStruct((N, H), F32)] * 4,
    )(h0, h1, w1at, w1bt, b1)


def _tc_edge_body(xa, xb, ef, w2at, w2bt, b2, g, be, out):
    x = xa[...].astype(F32) + xb[...].astype(F32)
    z = _ln_rows(x, g[...], be[...])
    z = jnp.maximum(z, 0.0)
    out[...] = (jnp.dot(z, w2at[...], preferred_element_type=F32)
                + jnp.dot(ef[...], w2bt[...], preferred_element_type=F32)
                + b2[...])


def _tc_edge(xa, xb, ef, w2at, w2bt, b2, g, be):
    grid = (E // BE,)
    bspec = pl.BlockSpec((BE, D), lambda i: (i, 0))
    nclass = 5
    return pl.pallas_call(
        _tc_edge_body,
        grid=grid,
        in_specs=[bspec, bspec,
                  pl.BlockSpec((BE, 2), lambda i: (i, 0)),
                  _full2((D, nclass)), _full2((2, nclass)),
                  _full2((1, nclass)),
                  _full2((1, D)), _full2((1, D))],
        out_specs=pl.BlockSpec((BE, nclass), lambda i: (i, 0)),
        out_shape=jax.ShapeDtypeStruct((E, nclass), F32),
    )(xa, xb, ef, w2at, w2bt, b2, g, be)


def kernel(h, edge_weight, edge_feat, params, edge_index):
    p = params
    src = edge_index[0]
    dst = edge_index[1]
    r1 = lambda v: v.reshape(1, -1)

    hh0, hh1 = _tc_proj(
        h, p['Wp0'].T, p['Wp1'].T,
        r1(p['cp0']), r1(p['cp1']), r1(p['gp0']), r1(p['gp1']),
        r1(p['betap0']), r1(p['betap1']))

    zrows = jnp.zeros((NPS, H), F32)
    zdeg = jnp.zeros((NDEG,), F32)
    ones = jnp.ones((BLK,), F32)

    w1 = p['W1']
    deg = None
    for l in range(3):
        if l == 0:
            agg0, agg1, degp = _sc_agg_deg(hh0, hh1, src, dst, edge_weight,
                                           zrows, zdeg, ones)
            deg = degp[:N].reshape(N, 1)
        else:
            agg0, agg1 = _sc_agg(hh0, hh1, src, dst, edge_weight,
                                 zrows, zdeg, ones)
        largs = (hh0, hh1, agg0, agg1, deg,
                 p[f'Wself{l}'].T, p[f'Wneigh{l}'].T,
                 r1(p[f'bself{l}']), r1(p[f'g{l}']), r1(p[f'beta{l}']))
        if l < 2:
            hh0, hh1 = _tc_layer(*largs)
        else:
            hh0, hh1, atab, btab = _tc_layerp(
                *largs, w1[:, :D].T, w1[:, D:].T, r1(p['b1']))

    pk = lambda t: lax.bitcast_convert_type(
        t.reshape(t.shape[0], -1, 2), jnp.int32)
    xai, xbi = _sc_pred(pk(atab), pk(btab), src, dst)
    upk = lambda t: lax.bitcast_convert_type(t, jnp.bfloat16).reshape(
        t.shape[0], -1)
    xa, xb = upk(xai), upk(xbi)

    w2 = p['W2']
    score = _tc_edge(xa, xb, edge_feat, w2[:, :D].T, w2[:, D:].T,
                     r1(p['b2']), r1(p['g_pred']), r1(p['beta_pred']))
    return score


# speedup vs baseline: 2.5843x; 2.5843x over previous
import functools

import jax
import jax.numpy as jnp
from jax import lax
from jax.experimental import pallas as pl
from jax.experimental.pallas import tpu as pltpu
from jax.experimental.pallas import tpu_sc as plsc

N = 10000
E = 160000
D = 256
H = 128
NC = 2
NS = 16
EPS = E // NS
BLK = 80
NBLK = EPS // BLK
NPAD = 10240
NPS = NPAD // NS
NDEG = 10240
F32 = jnp.float32

@functools.lru_cache(None)
def _get_mesh():
    return plsc.VectorSubcoreMesh(core_axis_name="c", subcore_axis_name="s",
                                  num_cores=NC, num_subcores=NS)


def _ln_rows(z, g, b, eps=1e-5):
    mu = jnp.mean(z, axis=-1, keepdims=True)
    var = jnp.mean((z - mu) ** 2, axis=-1, keepdims=True)
    return (z - mu) * jax.lax.rsqrt(var + eps) * g + b


@functools.lru_cache(None)
def _make_sc_agg(with_deg):
    out_type = [jax.ShapeDtypeStruct((NPAD, H), F32),
                jax.ShapeDtypeStruct((NPAD, H), F32)]
    if with_deg:
        out_type.append(jax.ShapeDtypeStruct((NDEG,), F32))

    scratch = dict(
        idx_s0=pltpu.VMEM((BLK,), jnp.int32),
        idx_s1=pltpu.VMEM((BLK,), jnp.int32),
        idx_d0=pltpu.VMEM((BLK,), jnp.int32),
        idx_d1=pltpu.VMEM((BLK,), jnp.int32),
        w_v0=pltpu.VMEM((BLK,), F32),
        w_v1=pltpu.VMEM((BLK,), F32),
        rows0=pltpu.VMEM((BLK, H), F32),
        rows1=pltpu.VMEM((BLK, H), F32),
        ones_v=pltpu.VMEM((BLK,), F32),
        agg_sp=pltpu.VMEM_SHARED((NPAD, H), F32),
        deg_sp=pltpu.VMEM_SHARED((NDEG,), F32),
        isem0=pltpu.SemaphoreType.DMA,
        isem1=pltpu.SemaphoreType.DMA,
        jsem0=pltpu.SemaphoreType.DMA,
        jsem1=pltpu.SemaphoreType.DMA,
        gsem0=pltpu.SemaphoreType.DMA,
        gsem1=pltpu.SemaphoreType.DMA,
        ssem0=pltpu.SemaphoreType.DMA,
        ssem1=pltpu.SemaphoreType.DMA,
        dsem0=pltpu.SemaphoreType.DMA,
        dsem1=pltpu.SemaphoreType.DMA,
    )

    @functools.partial(pl.kernel, mesh=_get_mesh(), out_type=out_type,
                       scratch_types=scratch)
    def sc_agg(hh0, hh1, src, dst, w, zrows, zdeg, ones, *refs,
               idx_s0, idx_s1, idx_d0, idx_d1, w_v0, w_v1, rows0, rows1,
               ones_v, agg_sp, deg_sp, isem0, isem1, jsem0, jsem1,
               gsem0, gsem1, ssem0, ssem1, dsem0, dsem1):
        if with_deg:
            agg0_o, agg1_o, deg_o = refs[0], refs[1], refs[2]
        else:
            agg0_o, agg1_o = refs[0], refs[1]
            deg_o = None

        c = lax.axis_index("c")
        s = lax.axis_index("s")
        idx_s = (idx_s0, idx_s1)
        idx_d = (idx_d0, idx_d1)
        w_v = (w_v0, w_v1)
        rows = (rows0, rows1)
        isem = (isem0, isem1)
        jsem = (jsem0, jsem1)
        gsem = (gsem0, gsem1)
        ssem = (ssem0, ssem1)
        dsem = (dsem0, dsem1)

        def run(tbl, agg_out, do_deg):
            pltpu.sync_copy(zrows, agg_sp.at[pl.ds(s * NPS, NPS)])
            if do_deg:
                pltpu.sync_copy(zdeg.at[pl.ds(s * 640, 640)],
                                deg_sp.at[pl.ds(s * 640, 640)])
                pltpu.sync_copy(ones, ones_v)
            plsc.subcore_barrier()

            def start_idx_sw(b, m):
                base = s * EPS + b * BLK
                pltpu.async_copy(src.at[pl.ds(base, BLK)], idx_s[m], isem[m])
                pltpu.async_copy(w.at[pl.ds(base, BLK)], w_v[m],
                                 isem[m])

            def wait_idx_sw(m):
                pltpu.make_async_copy(src.at[pl.ds(0, BLK)], idx_s[m],
                                      isem[m]).wait()
                pltpu.make_async_copy(w.at[pl.ds(0, BLK)], w_v[m],
                                      isem[m]).wait()

            def start_idx_d(b, m):
                base = s * EPS + b * BLK
                pltpu.async_copy(dst.at[pl.ds(base, BLK)], idx_d[m], jsem[m])

            def wait_idx_d(m):
                pltpu.make_async_copy(dst.at[pl.ds(0, BLK)], idx_d[m],
                                      jsem[m]).wait()

            def start_gather(m):
                pltpu.async_copy(tbl.at[idx_s[m]], rows[m], gsem[m])

            def wait_gather(m):
                pltpu.make_async_copy(tbl.at[idx_s[m]], rows[m],
                                      gsem[m]).wait()

            def start_scatter(m):
                pltpu.async_copy(rows[m], agg_sp.at[idx_d[m]], ssem[m],
                                 add=True)
                if do_deg:
                    pltpu.async_copy(ones_v, deg_sp.at[idx_d[m]], dsem[m],
                                     add=True)

            def wait_scatter(m):
                pltpu.make_async_copy(rows[m], agg_sp.at[idx_d[m]],
                                      ssem[m]).wait()
                if do_deg:
                    pltpu.make_async_copy(ones_v, deg_sp.at[idx_d[m]],
                                          dsem[m]).wait()

            def step(b, m):
                mo = 1 - m
                wait_gather(m)

                @pl.when(b + 1 < NBLK)
                def _():
                    @pl.when(b >= 1)
                    def _():
                        wait_scatter(mo)
                        start_idx_d(b + 1, mo)
                    wait_idx_sw(mo)
                    start_gather(mo)

                @plsc.parallel_loop(0, BLK, step=1, unroll=4)
                def _(i):
                    g = (i // 16) * 16
                    wchunk = w_v[m][pl.ds(g, 16)]
                    w16 = wchunk.at[jnp.zeros((16,), jnp.int32)
                                    + (i - g)].get(mode="promise_in_bounds")
                    for j in range(H // 16):
                        sl = pl.ds(j * 16, 16)
                        rows[m][i, sl] = rows[m][i, sl] * w16

                @pl.when(b + 2 < NBLK)
                def _():
                    start_idx_sw(b + 2, m)

                wait_idx_d(m)
                start_scatter(m)

            start_idx_sw(0, 0)
            start_idx_sw(1, 1)
            start_idx_d(0, 0)
            start_idx_d(1, 1)
            wait_idx_sw(0)
            start_gather(0)

            def pair_body(k, carry):
                step(2 * k, 0)
                step(2 * k + 1, 1)
                return carry
            lax.fori_loop(0, NBLK // 2, pair_body, 0)
            if NBLK % 2:
                step(NBLK - 1, 0)
            wait_scatter(1)
            wait_scatter(0)

            plsc.subcore_barrier()
            pltpu.sync_copy(agg_sp.at[pl.ds(s * NPS, NPS)],
                            agg_out.at[pl.ds(s * NPS, NPS)])
            if do_deg:
                pltpu.sync_copy(deg_sp.at[pl.ds(s * 640, 640)],
                                deg_o.at[pl.ds(s * 640, 640)])

        @pl.when(c == 0)
        def _():
            run(hh0, agg0_o, with_deg)

        @pl.when(c == 1)
        def _():
            run(hh1, agg1_o, False)

    return sc_agg


def _sc_agg_deg(*args):
    return _make_sc_agg(True)(*args)


def _sc_agg(*args):
    return _make_sc_agg(False)(*args)


@functools.lru_cache(None)
def _make_sc_pred():
    BLKP = 40
    EPW = E // (NC * NS)
    NBLKP = EPW // BLKP
    HP = H

    @functools.partial(
        pl.kernel, mesh=_get_mesh(),
        out_type=[jax.ShapeDtypeStruct((E, HP), jnp.int32),
                  jax.ShapeDtypeStruct((E, HP), jnp.int32)],
        scratch_types=dict(
            idx_s0=pltpu.VMEM((BLKP,), jnp.int32),
            idx_s1=pltpu.VMEM((BLKP,), jnp.int32),
            idx_d0=pltpu.VMEM((BLKP,), jnp.int32),
            idx_d1=pltpu.VMEM((BLKP,), jnp.int32),
            bufa0=pltpu.VMEM((BLKP, HP), jnp.int32),
            bufa1=pltpu.VMEM((BLKP, HP), jnp.int32),
            bufb0=pltpu.VMEM((BLKP, HP), jnp.int32),
            bufb1=pltpu.VMEM((BLKP, HP), jnp.int32),
            isem0=pltpu.SemaphoreType.DMA,
            isem1=pltpu.SemaphoreType.DMA,
            gsem0=pltpu.SemaphoreType.DMA,
            gsem1=pltpu.SemaphoreType.DMA,
            osem0=pltpu.SemaphoreType.DMA,
            osem1=pltpu.SemaphoreType.DMA,
        ),
    )
    def sc_pred(ta, tb, src, dst, xa_o, xb_o, *,
                idx_s0, idx_s1, idx_d0, idx_d1, bufa0, bufa1, bufb0, bufb1,
                isem0, isem1, gsem0, gsem1, osem0, osem1):
        c = lax.axis_index("c")
        s = lax.axis_index("s")
        wid = s * NC + c
        idx_s = (idx_s0, idx_s1)
        idx_d = (idx_d0, idx_d1)
        bufa = (bufa0, bufa1)
        bufb = (bufb0, bufb1)
        isem = (isem0, isem1)
        gsem = (gsem0, gsem1)
        osem = (osem0, osem1)

        def start_idx(b, m):
            base = wid * EPW + b * BLKP
            pltpu.async_copy(src.at[pl.ds(base, BLKP)], idx_s[m], isem[m])
            pltpu.async_copy(dst.at[pl.ds(base, BLKP)], idx_d[m], isem[m])

        def wait_idx(m):
            pltpu.make_async_copy(src.at[pl.ds(0, BLKP)], idx_s[m],
                                  isem[m]).wait()
            pltpu.make_async_copy(dst.at[pl.ds(0, BLKP)], idx_d[m],
                                  isem[m]).wait()

        def start_gather(m):
            pltpu.async_copy(ta.at[idx_s[m]], bufa[m], gsem[m])
            pltpu.async_copy(tb.at[idx_d[m]], bufb[m], gsem[m])

        def wait_gather(m):
            pltpu.make_async_copy(ta.at[idx_s[m]], bufa[m], gsem[m]).wait()
            pltpu.make_async_copy(tb.at[idx_d[m]], bufb[m], gsem[m]).wait()

        def start_out(b, m):
            base = wid * EPW + b * BLKP
            pltpu.async_copy(bufa[m], xa_o.at[pl.ds(base, BLKP)], osem[m])
            pltpu.async_copy(bufb[m], xb_o.at[pl.ds(base, BLKP)], osem[m])

        def wait_out(b, m):
            base = wid * EPW + b * BLKP
            pltpu.make_async_copy(bufa[m], xa_o.at[pl.ds(base, BLKP)],
                                  osem[m]).wait()
            pltpu.make_async_copy(bufb[m], xb_o.at[pl.ds(base, BLKP)],
                                  osem[m]).wait()

        def step(b, m):
            mo = 1 - m
            wait_gather(m)
            start_out(b, m)

            @pl.when(b + 1 < NBLKP)
            def _():
                wait_idx(mo)

                @pl.when(b >= 1)
                def _():
                    wait_out(b - 1, mo)
                start_gather(mo)

            @pl.when(b + 2 < NBLKP)
            def _():
                start_idx(b + 2, m)

        start_idx(0, 0)
        start_idx(1, 1)
        wait_idx(0)
        start_gather(0)

        def pair_body(k, carry):
            step(2 * k, 0)
            step(2 * k + 1, 1)
            return carry
        lax.fori_loop(0, NBLKP // 2, pair_body, 0)
        if NBLKP % 2:
            step(NBLKP - 1, 0)
        wait_out(NBLKP - 2, 1)
        wait_out(NBLKP - 1, 0)

    return sc_pred


def _sc_pred(*args):
    return _make_sc_pred()(*args)


BT = 1000
BE = 2000


def _full2(shape):
    return pl.BlockSpec(shape, lambda i: (0, 0))


def _tc_proj_body(h_ref, w0t, w1t, c0, c1, g0, g1, be0, be1, o0, o1):
    x = h_ref[...]
    for (lo, wt, cc, gg, bb, oo) in ((0, w0t, c0, g0, be0, o0),
                                     (H, w1t, c1, g1, be1, o1)):
        z = jnp.dot(x[:, lo:lo + H], wt[...],
                    preferred_element_type=F32) + cc[...]
        z = _ln_rows(z, gg[...], bb[...])
        oo[...] = jnp.maximum(z, 0.0)


def _tc_proj(h, w0t, w1t, c0, c1, g0, g1, be0, be1):
    grid = (N // BT,)
    return pl.pallas_call(
        _tc_proj_body,
        grid=grid,
        in_specs=[pl.BlockSpec((BT, D), lambda i: (i, 0)),
                  _full2((H, H)), _full2((H, H)),
                  _full2((1, H)), _full2((1, H)),
                  _full2((1, H)), _full2((1, H)),
                  _full2((1, H)), _full2((1, H))],
        out_specs=[pl.BlockSpec((BT, H), lambda i: (i, 0)),
                   pl.BlockSpec((BT, H), lambda i: (i, 0))],
        out_shape=[jax.ShapeDtypeStruct((N, H), F32),
                   jax.ShapeDtypeStruct((N, H), F32)],
    )(h, w0t, w1t, c0, c1, g0, g1, be0, be1)


def _tc_layer_body(h0, h1, a0, a1, deg, wst, wnt, bs, g, be, o0, o1):
    hcat = jnp.concatenate([h0[...], h1[...]], axis=1)
    dd = jnp.maximum(deg[...], 1.0)
    mean = jnp.concatenate([a0[...], a1[...]], axis=1) / dd
    rst = (jnp.dot(hcat, wst[...], preferred_element_type=F32) + bs[...]
           + jnp.dot(mean, wnt[...], preferred_element_type=F32))
    rst = jnp.maximum(rst, 0.0)
    z = _ln_rows(rst, g[...], be[...])
    o0[...] = z[:, :H]
    o1[...] = z[:, H:]


def _tc_layer(h0, h1, a0, a1, deg, wst, wnt, bs, g, be):
    grid = (N // BT,)
    bspec = pl.BlockSpec((BT, H), lambda i: (i, 0))
    return pl.pallas_call(
        _tc_layer_body,
        grid=grid,
        in_specs=[bspec, bspec, bspec, bspec,
                  pl.BlockSpec((BT, 1), lambda i: (i, 0)),
                  _full2((D, D)), _full2((D, D)),
                  _full2((1, D)), _full2((1, D)), _full2((1, D))],
        out_specs=[bspec, bspec],
        out_shape=[jax.ShapeDtypeStruct((N, H), F32),
                   jax.ShapeDtypeStruct((N, H), F32)],
    )(h0, h1, a0, a1, deg, wst, wnt, bs, g, be)


def _pack_halves(a):
    ab = a.astype(jnp.bfloat16).astype(F32)
    bits = lax.bitcast_convert_type(ab, jnp.int32)
    lo = lax.shift_right_logical(bits[:, :H], 16)
    hi = bits[:, H:] & jnp.int32(-65536)
    return lo | hi


def _unpack_halves(w):
    lo = lax.bitcast_convert_type(lax.shift_left(w, 16), F32)
    hi = lax.bitcast_convert_type(w & jnp.int32(-65536), F32)
    return jnp.concatenate([lo, hi], axis=1)


def _tc_layerp_body(h0, h1, a0, a1, deg, wst, wnt, bs, g, be,
                    w1at, w1bt, b1, o0, o1, ao, bo):
    hcat = jnp.concatenate([h0[...], h1[...]], axis=1)
    dd = jnp.maximum(deg[...], 1.0)
    mean = jnp.concatenate([a0[...], a1[...]], axis=1) / dd
    rst = (jnp.dot(hcat, wst[...], preferred_element_type=F32) + bs[...]
           + jnp.dot(mean, wnt[...], preferred_element_type=F32))
    rst = jnp.maximum(rst, 0.0)
    z = _ln_rows(rst, g[...], be[...])
    o0[...] = z[:, :H]
    o1[...] = z[:, H:]
    aa = jnp.dot(z, w1at[...], preferred_element_type=F32) + b1[...]
    bb = jnp.dot(z, w1bt[...], preferred_element_type=F32)
    ao[...] = _pack_halves(aa)
    bo[...] = _pack_halves(bb)


def _tc_layerp(h0, h1, a0, a1, deg, wst, wnt, bs, g, be, w1at, w1bt, b1):
    btp = 2000
    grid = (N // btp,)
    bspec = pl.BlockSpec((btp, H), lambda i: (i, 0))
    return pl.pallas_call(
        _tc_layerp_body,
        grid=grid,
        in_specs=[bspec, bspec, bspec, bspec,
                  pl.BlockSpec((btp, 1), lambda i: (i, 0)),
                  _full2((D, D)), _full2((D, D)),
                  _full2((1, D)), _full2((1, D)), _full2((1, D)),
                  _full2((D, D)), _full2((D, D)), _full2((1, D))],
        out_specs=[bspec, bspec, bspec, bspec],
        out_shape=([jax.ShapeDtypeStruct((N, H), F32)] * 2
                   + [jax.ShapeDtypeStruct((N, H), jnp.int32)] * 2),
    )(h0, h1, a0, a1, deg, wst, wnt, bs, g, be, w1at, w1bt, b1)


def _tc_nodemm_body(h0, h1, w1at, w1bt, b1, a0, a1, b0o, b1o):
    hcat = jnp.concatenate([h0[...], h1[...]], axis=1)
    a = jnp.dot(hcat, w1at[...], preferred_element_type=F32) + b1[...]
    b = jnp.dot(hcat, w1bt[...], preferred_element_type=F32)
    a0[...] = a[:, :H]
    a1[...] = a[:, H:]
    b0o[...] = b[:, :H]
    b1o[...] = b[:, H:]


def _tc_nodemm(h0, h1, w1at, w1bt, b1):
    grid = (N // BT,)
    bspec = pl.BlockSpec((BT, H), lambda i: (i, 0))
    return pl.pallas_call(
        _tc_nodemm_body,
        grid=grid,
        in_specs=[bspec, bspec, _full2((D, D)), _full2((D, D)),
                  _full2((1, D))],
        out_specs=[bspec, bspec, bspec, bspec],
        out_shape=[jax.ShapeDtypeStruct((N, H), F32)] * 4,
    )(h0, h1, w1at, w1bt, b1)


def _tc_edge_body(xa, xb, ef, w2at, w2bt, b2, g, be, out):
    x = _unpack_halves(xa[...]) + _unpack_halves(xb[...])
    z = _ln_rows(x, g[...], be[...])
    z = jnp.maximum(z, 0.0)
    out[...] = (jnp.dot(z, w2at[...], preferred_element_type=F32)
                + jnp.dot(ef[...], w2bt[...], preferred_element_type=F32)
                + b2[...])


def _tc_edge(xa, xb, ef, w2at, w2bt, b2, g, be):
    grid = (E // BE,)
    bspec = pl.BlockSpec((BE, H), lambda i: (i, 0))
    nclass = 5
    return pl.pallas_call(
        _tc_edge_body,
        grid=grid,
        in_specs=[bspec, bspec,
                  pl.BlockSpec((BE, 2), lambda i: (i, 0)),
                  _full2((D, nclass)), _full2((2, nclass)),
                  _full2((1, nclass)),
                  _full2((1, D)), _full2((1, D))],
        out_specs=pl.BlockSpec((BE, nclass), lambda i: (i, 0)),
        out_shape=jax.ShapeDtypeStruct((E, nclass), F32),
    )(xa, xb, ef, w2at, w2bt, b2, g, be)


def kernel(h, edge_weight, edge_feat, params, edge_index):
    p = params
    src = edge_index[0]
    dst = edge_index[1]
    r1 = lambda v: v.reshape(1, -1)

    hh0, hh1 = _tc_proj(
        h, p['Wp0'].T, p['Wp1'].T,
        r1(p['cp0']), r1(p['cp1']), r1(p['gp0']), r1(p['gp1']),
        r1(p['betap0']), r1(p['betap1']))

    zrows = jnp.zeros((NPS, H), F32)
    zdeg = jnp.zeros((NDEG,), F32)
    ones = jnp.ones((BLK,), F32)

    w1 = p['W1']
    deg = None
    for l in range(3):
        if l == 0:
            agg0, agg1, degp = _sc_agg_deg(hh0, hh1, src, dst, edge_weight,
                                           zrows, zdeg, ones)
            deg = degp[:N].reshape(N, 1)
        else:
            agg0, agg1 = _sc_agg(hh0, hh1, src, dst, edge_weight,
                                 zrows, zdeg, ones)
        largs = (hh0, hh1, agg0, agg1, deg,
                 p[f'Wself{l}'].T, p[f'Wneigh{l}'].T,
                 r1(p[f'bself{l}']), r1(p[f'g{l}']), r1(p[f'beta{l}']))
        if l < 2:
            hh0, hh1 = _tc_layer(*largs)
        else:
            hh0, hh1, atab, btab = _tc_layerp(
                *largs, w1[:, :D].T, w1[:, D:].T, r1(p['b1']))

    xa, xb = _sc_pred(atab, btab, src, dst)

    w2 = p['W2']
    score = _tc_edge(xa, xb, edge_feat, w2[:, :D].T, w2[:, D:].T,
                     r1(p['b2']), r1(p['g_pred']), r1(p['beta_pred']))
    return score


# speedup vs baseline: 2.6626x; 1.0303x over previous
import functools

import jax
import jax.numpy as jnp
from jax import lax
from jax.experimental import pallas as pl
from jax.experimental.pallas import tpu as pltpu
from jax.experimental.pallas import tpu_sc as plsc

N = 10000
E = 160000
D = 256
H = 128
NC = 2
NS = 16
EPS = E // NS
BLK = 80
NBLK = EPS // BLK
NPAD = 10240
NPS = NPAD // NS
NDEG = 10240
F32 = jnp.float32

@functools.lru_cache(None)
def _get_mesh():
    return plsc.VectorSubcoreMesh(core_axis_name="c", subcore_axis_name="s",
                                  num_cores=NC, num_subcores=NS)


def _ln_rows(z, g, b, eps=1e-5):
    mu = jnp.mean(z, axis=-1, keepdims=True)
    var = jnp.mean((z - mu) ** 2, axis=-1, keepdims=True)
    return (z - mu) * jax.lax.rsqrt(var + eps) * g + b


@functools.lru_cache(None)
def _make_sc_agg(with_deg):
    out_type = [jax.ShapeDtypeStruct((NPAD, H), F32),
                jax.ShapeDtypeStruct((NPAD, H), F32)]
    if with_deg:
        out_type.append(jax.ShapeDtypeStruct((NDEG,), F32))

    scratch = dict(
        idx_s0=pltpu.VMEM((BLK,), jnp.int32),
        idx_s1=pltpu.VMEM((BLK,), jnp.int32),
        idx_d0=pltpu.VMEM((BLK,), jnp.int32),
        idx_d1=pltpu.VMEM((BLK,), jnp.int32),
        w_v0=pltpu.VMEM((BLK,), F32),
        w_v1=pltpu.VMEM((BLK,), F32),
        rows0=pltpu.VMEM((BLK, H), F32),
        rows1=pltpu.VMEM((BLK, H), F32),
        ones_v=pltpu.VMEM((BLK,), F32),
        agg_sp=pltpu.VMEM_SHARED((NPAD, H), F32),
        deg_sp=pltpu.VMEM_SHARED((NDEG,), F32),
        isem0=pltpu.SemaphoreType.DMA,
        isem1=pltpu.SemaphoreType.DMA,
        jsem0=pltpu.SemaphoreType.DMA,
        jsem1=pltpu.SemaphoreType.DMA,
        gsem0=pltpu.SemaphoreType.DMA,
        gsem1=pltpu.SemaphoreType.DMA,
        ssem0=pltpu.SemaphoreType.DMA,
        ssem1=pltpu.SemaphoreType.DMA,
        dsem0=pltpu.SemaphoreType.DMA,
        dsem1=pltpu.SemaphoreType.DMA,
    )

    @functools.partial(pl.kernel, mesh=_get_mesh(), out_type=out_type,
                       scratch_types=scratch)
    def sc_agg(hh0, hh1, src, dst, w, zrows, zdeg, ones, *refs,
               idx_s0, idx_s1, idx_d0, idx_d1, w_v0, w_v1, rows0, rows1,
               ones_v, agg_sp, deg_sp, isem0, isem1, jsem0, jsem1,
               gsem0, gsem1, ssem0, ssem1, dsem0, dsem1):
        if with_deg:
            agg0_o, agg1_o, deg_o = refs[0], refs[1], refs[2]
        else:
            agg0_o, agg1_o = refs[0], refs[1]
            deg_o = None

        c = lax.axis_index("c")
        s = lax.axis_index("s")
        idx_s = (idx_s0, idx_s1)
        idx_d = (idx_d0, idx_d1)
        w_v = (w_v0, w_v1)
        rows = (rows0, rows1)
        isem = (isem0, isem1)
        jsem = (jsem0, jsem1)
        gsem = (gsem0, gsem1)
        ssem = (ssem0, ssem1)
        dsem = (dsem0, dsem1)

        def run(tbl, agg_out, do_deg):
            pltpu.sync_copy(zrows, agg_sp.at[pl.ds(s * NPS, NPS)])
            if do_deg:
                pltpu.sync_copy(zdeg.at[pl.ds(s * 640, 640)],
                                deg_sp.at[pl.ds(s * 640, 640)])
                pltpu.sync_copy(ones, ones_v)
            plsc.subcore_barrier()

            def start_idx_sw(b, m):
                base = s * EPS + b * BLK
                pltpu.async_copy(src.at[pl.ds(base, BLK)], idx_s[m], isem[m])
                pltpu.async_copy(w.at[pl.ds(base, BLK)], w_v[m],
                                 isem[m])

            def wait_idx_sw(m):
                pltpu.make_async_copy(src.at[pl.ds(0, BLK)], idx_s[m],
                                      isem[m]).wait()
                pltpu.make_async_copy(w.at[pl.ds(0, BLK)], w_v[m],
                                      isem[m]).wait()

            def start_idx_d(b, m):
                base = s * EPS + b * BLK
                pltpu.async_copy(dst.at[pl.ds(base, BLK)], idx_d[m], jsem[m])

            def wait_idx_d(m):
                pltpu.make_async_copy(dst.at[pl.ds(0, BLK)], idx_d[m],
                                      jsem[m]).wait()

            def start_gather(m):
                pltpu.async_copy(tbl.at[idx_s[m]], rows[m], gsem[m])

            def wait_gather(m):
                pltpu.make_async_copy(tbl.at[idx_s[m]], rows[m],
                                      gsem[m]).wait()

            def start_scatter(m):
                pltpu.async_copy(rows[m], agg_sp.at[idx_d[m]], ssem[m],
                                 add=True)
                if do_deg:
                    pltpu.async_copy(ones_v, deg_sp.at[idx_d[m]], dsem[m],
                                     add=True)

            def wait_scatter(m):
                pltpu.make_async_copy(rows[m], agg_sp.at[idx_d[m]],
                                      ssem[m]).wait()
                if do_deg:
                    pltpu.make_async_copy(ones_v, deg_sp.at[idx_d[m]],
                                          dsem[m]).wait()

            def step(b, m):
                mo = 1 - m
                wait_gather(m)

                @pl.when(b + 1 < NBLK)
                def _():
                    @pl.when(b >= 1)
                    def _():
                        wait_scatter(mo)
                        start_idx_d(b + 1, mo)
                    wait_idx_sw(mo)
                    start_gather(mo)

                @plsc.parallel_loop(0, BLK, step=1, unroll=4)
                def _(i):
                    g = (i // 16) * 16
                    wchunk = w_v[m][pl.ds(g, 16)]
                    w16 = wchunk.at[jnp.zeros((16,), jnp.int32)
                                    + (i - g)].get(mode="promise_in_bounds")
                    for j in range(H // 16):
                        sl = pl.ds(j * 16, 16)
                        rows[m][i, sl] = rows[m][i, sl] * w16

                @pl.when(b + 2 < NBLK)
                def _():
                    start_idx_sw(b + 2, m)

                wait_idx_d(m)
                start_scatter(m)

            start_idx_sw(0, 0)
            start_idx_sw(1, 1)
            start_idx_d(0, 0)
            start_idx_d(1, 1)
            wait_idx_sw(0)
            start_gather(0)

            def pair_body(k, carry):
                step(2 * k, 0)
                step(2 * k + 1, 1)
                return carry
            lax.fori_loop(0, NBLK // 2, pair_body, 0)
            if NBLK % 2:
                step(NBLK - 1, 0)
            wait_scatter(1)
            wait_scatter(0)

            plsc.subcore_barrier()
            pltpu.sync_copy(agg_sp.at[pl.ds(s * NPS, NPS)],
                            agg_out.at[pl.ds(s * NPS, NPS)])
            if do_deg:
                pltpu.sync_copy(deg_sp.at[pl.ds(s * 640, 640)],
                                deg_o.at[pl.ds(s * 640, 640)])

        @pl.when(c == 0)
        def _():
            run(hh0, agg0_o, with_deg)

        @pl.when(c == 1)
        def _():
            run(hh1, agg1_o, False)

    return sc_agg


def _sc_agg_deg(*args):
    return _make_sc_agg(True)(*args)


def _sc_agg(*args):
    return _make_sc_agg(False)(*args)


@functools.lru_cache(None)
def _make_sc_pred(eoff, ne):
    BLKP = 40
    EPW = ne // (NC * NS)
    NBLKP = EPW // BLKP
    HP = H

    @functools.partial(
        pl.kernel, mesh=_get_mesh(),
        out_type=[jax.ShapeDtypeStruct((ne, HP), jnp.int32),
                  jax.ShapeDtypeStruct((ne, HP), jnp.int32)],
        scratch_types=dict(
            idx_s0=pltpu.VMEM((BLKP,), jnp.int32),
            idx_s1=pltpu.VMEM((BLKP,), jnp.int32),
            idx_d0=pltpu.VMEM((BLKP,), jnp.int32),
            idx_d1=pltpu.VMEM((BLKP,), jnp.int32),
            bufa0=pltpu.VMEM((BLKP, HP), jnp.int32),
            bufa1=pltpu.VMEM((BLKP, HP), jnp.int32),
            bufb0=pltpu.VMEM((BLKP, HP), jnp.int32),
            bufb1=pltpu.VMEM((BLKP, HP), jnp.int32),
            isem0=pltpu.SemaphoreType.DMA,
            isem1=pltpu.SemaphoreType.DMA,
            gsem0=pltpu.SemaphoreType.DMA,
            gsem1=pltpu.SemaphoreType.DMA,
            osem0=pltpu.SemaphoreType.DMA,
            osem1=pltpu.SemaphoreType.DMA,
        ),
    )
    def sc_pred(ta, tb, src, dst, xa_o, xb_o, *,
                idx_s0, idx_s1, idx_d0, idx_d1, bufa0, bufa1, bufb0, bufb1,
                isem0, isem1, gsem0, gsem1, osem0, osem1):
        c = lax.axis_index("c")
        s = lax.axis_index("s")
        wid = s * NC + c
        idx_s = (idx_s0, idx_s1)
        idx_d = (idx_d0, idx_d1)
        bufa = (bufa0, bufa1)
        bufb = (bufb0, bufb1)
        isem = (isem0, isem1)
        gsem = (gsem0, gsem1)
        osem = (osem0, osem1)

        def start_idx(b, m):
            base = eoff + wid * EPW + b * BLKP
            pltpu.async_copy(src.at[pl.ds(base, BLKP)], idx_s[m], isem[m])
            pltpu.async_copy(dst.at[pl.ds(base, BLKP)], idx_d[m], isem[m])

        def wait_idx(m):
            pltpu.make_async_copy(src.at[pl.ds(0, BLKP)], idx_s[m],
                                  isem[m]).wait()
            pltpu.make_async_copy(dst.at[pl.ds(0, BLKP)], idx_d[m],
                                  isem[m]).wait()

        def start_gather(m):
            pltpu.async_copy(ta.at[idx_s[m]], bufa[m], gsem[m])
            pltpu.async_copy(tb.at[idx_d[m]], bufb[m], gsem[m])

        def wait_gather(m):
            pltpu.make_async_copy(ta.at[idx_s[m]], bufa[m], gsem[m]).wait()
            pltpu.make_async_copy(tb.at[idx_d[m]], bufb[m], gsem[m]).wait()

        def start_out(b, m):
            base = wid * EPW + b * BLKP
            pltpu.async_copy(bufa[m], xa_o.at[pl.ds(base, BLKP)], osem[m])
            pltpu.async_copy(bufb[m], xb_o.at[pl.ds(base, BLKP)], osem[m])

        def wait_out(b, m):
            base = wid * EPW + b * BLKP
            pltpu.make_async_copy(bufa[m], xa_o.at[pl.ds(base, BLKP)],
                                  osem[m]).wait()
            pltpu.make_async_copy(bufb[m], xb_o.at[pl.ds(base, BLKP)],
                                  osem[m]).wait()

        def step(b, m):
            mo = 1 - m
            wait_gather(m)
            start_out(b, m)

            @pl.when(b + 1 < NBLKP)
            def _():
                wait_idx(mo)

                @pl.when(b >= 1)
                def _():
                    wait_out(b - 1, mo)
                start_gather(mo)

            @pl.when(b + 2 < NBLKP)
            def _():
                start_idx(b + 2, m)

        start_idx(0, 0)
        start_idx(1, 1)
        wait_idx(0)
        start_gather(0)

        def pair_body(k, carry):
            step(2 * k, 0)
            step(2 * k + 1, 1)
            return carry
        lax.fori_loop(0, NBLKP // 2, pair_body, 0)
        if NBLKP % 2:
            step(NBLKP - 1, 0)
        wait_out(NBLKP - 2, 1)
        wait_out(NBLKP - 1, 0)

    return sc_pred


def _sc_pred(eoff, ne, *args):
    return _make_sc_pred(eoff, ne)(*args)


BT = 1000
BE = 2000


def _full2(shape):
    return pl.BlockSpec(shape, lambda i: (0, 0))


def _tc_proj_body(h_ref, w0t, w1t, c0, c1, g0, g1, be0, be1, o0, o1):
    x = h_ref[...]
    for (lo, wt, cc, gg, bb, oo) in ((0, w0t, c0, g0, be0, o0),
                                     (H, w1t, c1, g1, be1, o1)):
        z = jnp.dot(x[:, lo:lo + H], wt[...],
                    preferred_element_type=F32) + cc[...]
        z = _ln_rows(z, gg[...], bb[...])
        oo[...] = jnp.maximum(z, 0.0)


def _tc_proj(h, w0t, w1t, c0, c1, g0, g1, be0, be1):
    grid = (N // BT,)
    return pl.pallas_call(
        _tc_proj_body,
        grid=grid,
        in_specs=[pl.BlockSpec((BT, D), lambda i: (i, 0)),
                  _full2((H, H)), _full2((H, H)),
                  _full2((1, H)), _full2((1, H)),
                  _full2((1, H)), _full2((1, H)),
                  _full2((1, H)), _full2((1, H))],
        out_specs=[pl.BlockSpec((BT, H), lambda i: (i, 0)),
                   pl.BlockSpec((BT, H), lambda i: (i, 0))],
        out_shape=[jax.ShapeDtypeStruct((N, H), F32),
                   jax.ShapeDtypeStruct((N, H), F32)],
    )(h, w0t, w1t, c0, c1, g0, g1, be0, be1)


def _tc_layer_body(h0, h1, a0, a1, deg, wst, wnt, bs, g, be, o0, o1):
    hcat = jnp.concatenate([h0[...], h1[...]], axis=1)
    dd = jnp.maximum(deg[...], 1.0)
    mean = jnp.concatenate([a0[...], a1[...]], axis=1) / dd
    rst = (jnp.dot(hcat, wst[...], preferred_element_type=F32) + bs[...]
           + jnp.dot(mean, wnt[...], preferred_element_type=F32))
    rst = jnp.maximum(rst, 0.0)
    z = _ln_rows(rst, g[...], be[...])
    o0[...] = z[:, :H]
    o1[...] = z[:, H:]


def _tc_layer(h0, h1, a0, a1, deg, wst, wnt, bs, g, be):
    grid = (N // BT,)
    bspec = pl.BlockSpec((BT, H), lambda i: (i, 0))
    return pl.pallas_call(
        _tc_layer_body,
        grid=grid,
        in_specs=[bspec, bspec, bspec, bspec,
                  pl.BlockSpec((BT, 1), lambda i: (i, 0)),
                  _full2((D, D)), _full2((D, D)),
                  _full2((1, D)), _full2((1, D)), _full2((1, D))],
        out_specs=[bspec, bspec],
        out_shape=[jax.ShapeDtypeStruct((N, H), F32),
                   jax.ShapeDtypeStruct((N, H), F32)],
    )(h0, h1, a0, a1, deg, wst, wnt, bs, g, be)


def _pack_halves(a):
    ab = a.astype(jnp.bfloat16).astype(F32)
    bits = lax.bitcast_convert_type(ab, jnp.int32)
    lo = lax.shift_right_logical(bits[:, :H], 16)
    hi = bits[:, H:] & jnp.int32(-65536)
    return lo | hi


def _unpack_halves(w):
    lo = lax.bitcast_convert_type(lax.shift_left(w, 16), F32)
    hi = lax.bitcast_convert_type(w & jnp.int32(-65536), F32)
    return jnp.concatenate([lo, hi], axis=1)


def _tc_layerp_body(h0, h1, a0, a1, deg, wst, wnt, bs, g, be,
                    w1at, w1bt, b1, o0, o1, ao, bo):
    hcat = jnp.concatenate([h0[...], h1[...]], axis=1)
    dd = jnp.maximum(deg[...], 1.0)
    mean = jnp.concatenate([a0[...], a1[...]], axis=1) / dd
    rst = (jnp.dot(hcat, wst[...], preferred_element_type=F32) + bs[...]
           + jnp.dot(mean, wnt[...], preferred_element_type=F32))
    rst = jnp.maximum(rst, 0.0)
    z = _ln_rows(rst, g[...], be[...])
    o0[...] = z[:, :H]
    o1[...] = z[:, H:]
    aa = jnp.dot(z, w1at[...], preferred_element_type=F32) + b1[...]
    bb = jnp.dot(z, w1bt[...], preferred_element_type=F32)
    ao[...] = _pack_halves(aa)
    bo[...] = _pack_halves(bb)


def _tc_layerp(h0, h1, a0, a1, deg, wst, wnt, bs, g, be, w1at, w1bt, b1):
    btp = 2000
    grid = (N // btp,)
    bspec = pl.BlockSpec((btp, H), lambda i: (i, 0))
    return pl.pallas_call(
        _tc_layerp_body,
        grid=grid,
        in_specs=[bspec, bspec, bspec, bspec,
                  pl.BlockSpec((btp, 1), lambda i: (i, 0)),
                  _full2((D, D)), _full2((D, D)),
                  _full2((1, D)), _full2((1, D)), _full2((1, D)),
                  _full2((D, D)), _full2((D, D)), _full2((1, D))],
        out_specs=[bspec, bspec, bspec, bspec],
        out_shape=([jax.ShapeDtypeStruct((N, H), F32)] * 2
                   + [jax.ShapeDtypeStruct((N, H), jnp.int32)] * 2),
    )(h0, h1, a0, a1, deg, wst, wnt, bs, g, be, w1at, w1bt, b1)


def _tc_nodemm_body(h0, h1, w1at, w1bt, b1, a0, a1, b0o, b1o):
    hcat = jnp.concatenate([h0[...], h1[...]], axis=1)
    a = jnp.dot(hcat, w1at[...], preferred_element_type=F32) + b1[...]
    b = jnp.dot(hcat, w1bt[...], preferred_element_type=F32)
    a0[...] = a[:, :H]
    a1[...] = a[:, H:]
    b0o[...] = b[:, :H]
    b1o[...] = b[:, H:]


def _tc_nodemm(h0, h1, w1at, w1bt, b1):
    grid = (N // BT,)
    bspec = pl.BlockSpec((BT, H), lambda i: (i, 0))
    return pl.pallas_call(
        _tc_nodemm_body,
        grid=grid,
        in_specs=[bspec, bspec, _full2((D, D)), _full2((D, D)),
                  _full2((1, D))],
        out_specs=[bspec, bspec, bspec, bspec],
        out_shape=[jax.ShapeDtypeStruct((N, H), F32)] * 4,
    )(h0, h1, w1at, w1bt, b1)


def _tc_edge_body(xa, xb, ef, w2at, w2bt, b2, g, be, out):
    x = _unpack_halves(xa[...]) + _unpack_halves(xb[...])
    z = _ln_rows(x, g[...], be[...])
    z = jnp.maximum(z, 0.0)
    out[...] = (jnp.dot(z, w2at[...], preferred_element_type=F32)
                + jnp.dot(ef[...], w2bt[...], preferred_element_type=F32)
                + b2[...])


def _tc_edge(xa, xb, ef, w2at, w2bt, b2, g, be):
    ne = xa.shape[0]
    grid = (ne // BE,)
    bspec = pl.BlockSpec((BE, H), lambda i: (i, 0))
    nclass = 5
    return pl.pallas_call(
        _tc_edge_body,
        grid=grid,
        in_specs=[bspec, bspec,
                  pl.BlockSpec((BE, 2), lambda i: (i, 0)),
                  _full2((D, nclass)), _full2((2, nclass)),
                  _full2((1, nclass)),
                  _full2((1, D)), _full2((1, D))],
        out_specs=pl.BlockSpec((BE, nclass), lambda i: (i, 0)),
        out_shape=jax.ShapeDtypeStruct((ne, nclass), F32),
    )(xa, xb, ef, w2at, w2bt, b2, g, be)


def kernel(h, edge_weight, edge_feat, params, edge_index):
    p = params
    src = edge_index[0]
    dst = edge_index[1]
    r1 = lambda v: v.reshape(1, -1)

    hh0, hh1 = _tc_proj(
        h, p['Wp0'].T, p['Wp1'].T,
        r1(p['cp0']), r1(p['cp1']), r1(p['gp0']), r1(p['gp1']),
        r1(p['betap0']), r1(p['betap1']))

    zrows = jnp.zeros((NPS, H), F32)
    zdeg = jnp.zeros((NDEG,), F32)
    ones = jnp.ones((BLK,), F32)

    w1 = p['W1']
    deg = None
    for l in range(3):
        if l == 0:
            agg0, agg1, degp = _sc_agg_deg(hh0, hh1, src, dst, edge_weight,
                                           zrows, zdeg, ones)
            deg = degp[:N].reshape(N, 1)
        else:
            agg0, agg1 = _sc_agg(hh0, hh1, src, dst, edge_weight,
                                 zrows, zdeg, ones)
        largs = (hh0, hh1, agg0, agg1, deg,
                 p[f'Wself{l}'].T, p[f'Wneigh{l}'].T,
                 r1(p[f'bself{l}']), r1(p[f'g{l}']), r1(p[f'beta{l}']))
        if l < 2:
            hh0, hh1 = _tc_layer(*largs)
        else:
            hh0, hh1, atab, btab = _tc_layerp(
                *largs, w1[:, :D].T, w1[:, D:].T, r1(p['b1']))

    w2 = p['W2']
    eargs = (w2[:, :D].T, w2[:, D:].T,
             r1(p['b2']), r1(p['g_pred']), r1(p['beta_pred']))
    ea = 96000
    xa_a, xb_a = _sc_pred(0, ea, atab, btab, src, dst)
    xa_b, xb_b = _sc_pred(ea, E - ea, atab, btab, src, dst)
    score_a = _tc_edge(xa_a, xb_a, edge_feat[:ea], *eargs)
    score_b = _tc_edge(xa_b, xb_b, edge_feat[ea:], *eargs)
    return jnp.concatenate([score_a, score_b], axis=0)
